# initial kernel scaffold (unmeasured)
import jax
import jax.numpy as jnp
from jax import lax
from jax.experimental import pallas as pl
from jax.experimental.pallas import tpu as pltpu


def kernel(Q, K, V):
    b, sq, h, d = Q.shape
    bh = b * h
    scale = d ** -0.5

    Qf = Q.transpose(0, 2, 1, 3).reshape(bh, sq, d)
    Kf = K.transpose(0, 2, 1, 3).reshape(bh, sq, d)
    Vf = V.transpose(0, 2, 1, 3).reshape(bh, sq, d)

    def body(qf_ref, kf_ref, vf_ref, out_ref,
             k_oth, v_oth, send_sems, recv_sems):
        my_x = lax.axis_index("x")
        my_y = lax.axis_index("y")
        my_z = lax.axis_index("z")
        partner = (1 - my_x, my_y, my_z)

        barrier_sem = pltpu.get_barrier_semaphore()
        pl.semaphore_signal(
            barrier_sem, inc=1,
            device_id=partner, device_id_type=pl.DeviceIdType.MESH,
        )
        pl.semaphore_wait(barrier_sem, 1)

        rdma_k = pltpu.make_async_remote_copy(
            src_ref=kf_ref, dst_ref=k_oth,
            send_sem=send_sems.at[0], recv_sem=recv_sems.at[0],
            device_id=partner, device_id_type=pl.DeviceIdType.MESH,
        )
        rdma_v = pltpu.make_async_remote_copy(
            src_ref=vf_ref, dst_ref=v_oth,
            send_sem=send_sems.at[1], recv_sem=recv_sems.at[1],
            device_id=partner, device_id_type=pl.DeviceIdType.MESH,
        )
        rdma_k.start()
        rdma_v.start()
        rdma_k.wait()
        rdma_v.wait()

        def one(i, _):
            q = qf_ref[i, :, :] * scale
            k_l = kf_ref[i, :, :]
            v_l = vf_ref[i, :, :]
            k_o = k_oth[i, :, :]
            v_o = v_oth[i, :, :]
            s_l = jax.lax.dot_general(
                q, k_l, (((1,), (1,)), ((), ())),
                preferred_element_type=jnp.float32)
            s_o = jax.lax.dot_general(
                q, k_o, (((1,), (1,)), ((), ())),
                preferred_element_type=jnp.float32)
            m = jnp.maximum(jnp.max(s_l, axis=1, keepdims=True),
                            jnp.max(s_o, axis=1, keepdims=True))
            e_l = jnp.exp(s_l - m)
            e_o = jnp.exp(s_o - m)
            denom = (jnp.sum(e_l, axis=1, keepdims=True)
                     + jnp.sum(e_o, axis=1, keepdims=True))
            o = (jax.lax.dot_general(
                     e_l, v_l, (((1,), (0,)), ((), ())),
                     preferred_element_type=jnp.float32)
                 + jax.lax.dot_general(
                     e_o, v_o, (((1,), (0,)), ((), ())),
                     preferred_element_type=jnp.float32))
            out_ref[i, :, :] = o / denom
            return 0

        lax.fori_loop(0, bh, one, 0)

    out = pl.pallas_call(
        body,
        out_shape=jax.ShapeDtypeStruct((bh, sq, d), jnp.float32),
        in_specs=[pl.BlockSpec(memory_space=pltpu.VMEM)] * 3,
        out_specs=pl.BlockSpec(memory_space=pltpu.VMEM),
        scratch_shapes=[
            pltpu.VMEM((bh, sq, d), jnp.float32),
            pltpu.VMEM((bh, sq, d), jnp.float32),
            pltpu.SemaphoreType.DMA((2,)),
            pltpu.SemaphoreType.DMA((2,)),
        ],
        compiler_params=pltpu.CompilerParams(collective_id=0),
    )(Qf, Kf, Vf)

    return out.reshape(b, h, sq, d).transpose(0, 2, 1, 3)


# baseline (device time: 255097 ns/iter reference)
import jax
import jax.numpy as jnp
from jax import lax
from jax.experimental import pallas as pl
from jax.experimental.pallas import tpu as pltpu


def kernel(Q, K, V):
    b, sq, h, d = Q.shape
    bh = b * h
    scale = d ** -0.5

    Qf = Q.transpose(0, 2, 1, 3).reshape(bh, sq, d)
    Kf = K.transpose(0, 2, 1, 3).reshape(bh, sq, d)
    Vf = V.transpose(0, 2, 1, 3).reshape(bh, sq, d)

    def body(qf_ref, kf_ref, vf_ref, out_ref,
             k_oth, v_oth, send_sems, recv_sems):
        my_x = lax.axis_index("x")
        my_y = lax.axis_index("y")
        my_z = lax.axis_index("z")
        partner = (1 - my_x, my_y, my_z)

        barrier_sem = pltpu.get_barrier_semaphore()
        pl.semaphore_signal(
            barrier_sem, inc=1,
            device_id=partner, device_id_type=pl.DeviceIdType.MESH,
        )
        pl.semaphore_wait(barrier_sem, 1)

        rdma_k = pltpu.make_async_remote_copy(
            src_ref=kf_ref, dst_ref=k_oth,
            send_sem=send_sems.at[0], recv_sem=recv_sems.at[0],
            device_id=partner, device_id_type=pl.DeviceIdType.MESH,
        )
        rdma_v = pltpu.make_async_remote_copy(
            src_ref=vf_ref, dst_ref=v_oth,
            send_sem=send_sems.at[1], recv_sem=recv_sems.at[1],
            device_id=partner, device_id_type=pl.DeviceIdType.MESH,
        )
        rdma_k.start()
        rdma_v.start()
        rdma_k.wait()
        rdma_v.wait()

        def one(i, _):
            q = qf_ref[i, :, :] * scale
            k_l = kf_ref[i, :, :]
            v_l = vf_ref[i, :, :]
            k_o = k_oth[i, :, :]
            v_o = v_oth[i, :, :]
            s_l = jax.lax.dot_general(
                q, k_l, (((1,), (1,)), ((), ())),
                preferred_element_type=jnp.float32)
            s_o = jax.lax.dot_general(
                q, k_o, (((1,), (1,)), ((), ())),
                preferred_element_type=jnp.float32)
            m = jnp.maximum(jnp.max(s_l, axis=1, keepdims=True),
                            jnp.max(s_o, axis=1, keepdims=True))
            e_l = jnp.exp(s_l - m)
            e_o = jnp.exp(s_o - m)
            denom = (jnp.sum(e_l, axis=1, keepdims=True)
                     + jnp.sum(e_o, axis=1, keepdims=True))
            o = (jax.lax.dot_general(
                     e_l, v_l, (((1,), (0,)), ((), ())),
                     preferred_element_type=jnp.float32)
                 + jax.lax.dot_general(
                     e_o, v_o, (((1,), (0,)), ((), ())),
                     preferred_element_type=jnp.float32))
            out_ref[i, :, :] = o / denom
            return 0

        lax.fori_loop(0, bh, one, 0)

    out = pl.pallas_call(
        body,
        out_shape=jax.ShapeDtypeStruct((bh, sq, d), jnp.float32),
        in_specs=[pl.BlockSpec(memory_space=pltpu.VMEM)] * 3,
        out_specs=pl.BlockSpec(memory_space=pltpu.VMEM),
        scratch_shapes=[
            pltpu.VMEM((bh, sq, d), jnp.float32),
            pltpu.VMEM((bh, sq, d), jnp.float32),
            pltpu.SemaphoreType.DMA((2,)),
            pltpu.SemaphoreType.DMA((2,)),
        ],
        compiler_params=pltpu.CompilerParams(
            collective_id=0, vmem_limit_bytes=64 * 1024 * 1024),
    )(Qf, Kf, Vf)

    return out.reshape(b, h, sq, d).transpose(0, 2, 1, 3)


# device time: 45619 ns/iter; 5.5919x vs baseline; 5.5919x over previous
import jax
import jax.numpy as jnp
from jax import lax
from jax.experimental import pallas as pl
from jax.experimental.pallas import tpu as pltpu

N_CHUNK = 4


def kernel(Q, K, V):
    b, sq, h, d = Q.shape
    bh = b * h
    quart = bh // 4
    g = quart // N_CHUNK
    scale = d ** -0.5

    zp_out = lax.axis_index("z") % 2
    qi_out = 2 * lax.axis_index("y") + zp_out
    Qh = lax.dynamic_slice_in_dim(Q, qi_out, 1, 0)
    Kh = lax.dynamic_slice_in_dim(K, qi_out, 1, 0)
    Vh = lax.dynamic_slice_in_dim(V, qi_out, 1, 0)
    Qf = (Qh * scale).astype(jnp.bfloat16).transpose(0, 2, 1, 3).reshape(quart, sq, d)
    KT = Kh.astype(jnp.bfloat16).transpose(0, 2, 3, 1).reshape(quart, d, sq)
    VT = Vh.astype(jnp.bfloat16).transpose(0, 2, 3, 1).reshape(quart, d, sq)

    def body(qf_ref, kt_ref, vt_ref, out_ref,
             k_rem, v_rem, sx, rx, sor, sol, sfw, rlo, rlf, rro):
        my_x = lax.axis_index("x")
        my_y = lax.axis_index("y")
        my_z = lax.axis_index("z")
        zp = my_z % 2
        px = (1 - my_x, my_y, my_z)

        podd = (my_y + zp) % 2
        zflip = my_z + 1 - 2 * zp
        r_y = jnp.where(podd == 0, 1 - my_y, my_y)
        r_z = jnp.where(podd == 0, my_z, zflip)
        l_y = jnp.where(podd == 1, 1 - my_y, my_y)
        l_z = jnp.where(podd == 1, my_z, zflip)
        right = (my_x, r_y, r_z)
        left = (my_x, l_y, l_z)

        qi = 2 * my_y + zp
        qi_left = 2 * l_y + (l_z % 2)
        qi_opp = 2 * (1 - my_y) + (1 - zp)
        my_base = qi * quart

        barrier_sem = pltpu.get_barrier_semaphore()
        for nbr in (px, left, right):
            pl.semaphore_signal(
                barrier_sem, inc=1,
                device_id=nbr, device_id_type=pl.DeviceIdType.MESH,
            )
        pl.semaphore_wait(barrier_sem, 3)

        def compute_chunk(lc):
            def one(j, _):
                i = lc + j
                gi = my_base + i
                q = qf_ref[i, :, :]
                s_l = jax.lax.dot_general(
                    q, kt_ref[i, :, :], (((1,), (0,)), ((), ())),
                    preferred_element_type=jnp.float32)
                s_o = jax.lax.dot_general(
                    q, k_rem[i, :, :], (((1,), (0,)), ((), ())),
                    preferred_element_type=jnp.float32)
                m = jnp.maximum(jnp.max(s_l, axis=1, keepdims=True),
                                jnp.max(s_o, axis=1, keepdims=True))
                e_l = jnp.exp(s_l - m)
                e_o = jnp.exp(s_o - m)
                r = 1.0 / (jnp.sum(e_l, axis=1, keepdims=True)
                           + jnp.sum(e_o, axis=1, keepdims=True))
                p_l = (e_l * r).astype(jnp.bfloat16)
                p_o = (e_o * r).astype(jnp.bfloat16)
                ot = (jax.lax.dot_general(
                          vt_ref[i, :, :], p_l, (((1,), (1,)), ((), ())),
                          preferred_element_type=jnp.float32)
                      + jax.lax.dot_general(
                          v_rem[i, :, :], p_o, (((1,), (1,)), ((), ())),
                          preferred_element_type=jnp.float32))
                out_ref[gi, :, :] = ot.astype(jnp.bfloat16)
                return 0

            lax.fori_loop(0, g, one, 0)

        x_rdmas = []
        for c in range(N_CHUNK):
            sl = pl.ds(c * g, g)
            rk = pltpu.make_async_remote_copy(
                src_ref=kt_ref.at[sl], dst_ref=k_rem.at[sl],
                send_sem=sx.at[0, c], recv_sem=rx.at[0, c],
                device_id=px, device_id_type=pl.DeviceIdType.MESH,
            )
            rv = pltpu.make_async_remote_copy(
                src_ref=vt_ref.at[sl], dst_ref=v_rem.at[sl],
                send_sem=sx.at[1, c], recv_sem=rx.at[1, c],
                device_id=px, device_id_type=pl.DeviceIdType.MESH,
            )
            rk.start()
            rv.start()
            x_rdmas.append((rk, rv))

        drain = []
        for c in range(N_CHUNK):
            rk, rv = x_rdmas[c]
            rk.wait_recv()
            rv.wait_recv()
            compute_chunk(c * g)
            osl = pl.ds(my_base + c * g, g)
            to_r = pltpu.make_async_remote_copy(
                src_ref=out_ref.at[osl], dst_ref=out_ref.at[osl],
                send_sem=sor.at[c], recv_sem=rlo.at[c],
                device_id=right, device_id_type=pl.DeviceIdType.MESH,
            )
            to_l = pltpu.make_async_remote_copy(
                src_ref=out_ref.at[osl], dst_ref=out_ref.at[osl],
                send_sem=sol.at[c], recv_sem=rro.at[c],
                device_id=left, device_id_type=pl.DeviceIdType.MESH,
            )
            to_r.start()
            to_l.start()
            lsl = pl.ds(qi_left * quart + c * g, g)
            lrecv = pltpu.make_async_remote_copy(
                src_ref=out_ref.at[lsl], dst_ref=out_ref.at[lsl],
                send_sem=sor.at[c], recv_sem=rlo.at[c],
                device_id=left, device_id_type=pl.DeviceIdType.MESH,
            )
            lrecv.wait_recv()
            fwd = pltpu.make_async_remote_copy(
                src_ref=out_ref.at[lsl], dst_ref=out_ref.at[lsl],
                send_sem=sfw.at[c], recv_sem=rlf.at[c],
                device_id=right, device_id_type=pl.DeviceIdType.MESH,
            )
            fwd.start()
            drain.extend([to_r, to_l, fwd])

        for c in range(N_CHUNK):
            rsl = pl.ds(qi_opp * quart + c * g, g)
            wf = pltpu.make_async_remote_copy(
                src_ref=out_ref.at[rsl], dst_ref=out_ref.at[rsl],
                send_sem=sfw.at[c], recv_sem=rlf.at[c],
                device_id=left, device_id_type=pl.DeviceIdType.MESH,
            )
            wf.wait_recv()
        for c in range(N_CHUNK):
            qr = 6 - qi - qi_left - qi_opp
            rsl = pl.ds(qr * quart + c * g, g)
            wr = pltpu.make_async_remote_copy(
                src_ref=out_ref.at[rsl], dst_ref=out_ref.at[rsl],
                send_sem=sol.at[c], recv_sem=rro.at[c],
                device_id=right, device_id_type=pl.DeviceIdType.MESH,
            )
            wr.wait_recv()

        for rk, rv in x_rdmas:
            rk.wait_send()
            rv.wait_send()
        for rr in drain:
            rr.wait_send()

    out = pl.pallas_call(
        body,
        out_shape=jax.ShapeDtypeStruct((bh, d, sq), jnp.bfloat16),
        in_specs=[pl.BlockSpec(memory_space=pltpu.VMEM)] * 3,
        out_specs=pl.BlockSpec(memory_space=pltpu.VMEM),
        scratch_shapes=[
            pltpu.VMEM((quart, d, sq), jnp.bfloat16),
            pltpu.VMEM((quart, d, sq), jnp.bfloat16),
            pltpu.SemaphoreType.DMA((2, N_CHUNK)),
            pltpu.SemaphoreType.DMA((2, N_CHUNK)),
            pltpu.SemaphoreType.DMA((N_CHUNK,)),
            pltpu.SemaphoreType.DMA((N_CHUNK,)),
            pltpu.SemaphoreType.DMA((N_CHUNK,)),
            pltpu.SemaphoreType.DMA((N_CHUNK,)),
            pltpu.SemaphoreType.DMA((N_CHUNK,)),
            pltpu.SemaphoreType.DMA((N_CHUNK,)),
        ],
        compiler_params=pltpu.CompilerParams(
            collective_id=0, vmem_limit_bytes=64 * 1024 * 1024),
    )(Qf, KT, VT)

    return (out.reshape(b, h, d, sq).transpose(0, 3, 1, 2)
            .astype(jnp.float32))
